# Initial kernel scaffold; baseline (speedup 1.0000x reference)
import jax
import jax.numpy as jnp
from jax.experimental import pallas as pl


def kernel(h_V, h_E, P_idx, batch_id, params):
    raise NotImplementedError("write your pallas kernel here")


# speedup vs baseline: 20.7588x; 20.7588x over previous
import functools
import math

import jax
import jax.numpy as jnp
from jax import lax
from jax.experimental import pallas as pl
from jax.experimental.pallas import tpu as pltpu
from jax.experimental.pallas import tpu_sc as plsc

N = 10000
E = 320000
HID = 128
NIN = 256
HEADS = 4
DH = HID // HEADS
NG = 16

_CHUNK = 128
_NCHUNK = E // _CHUNK
_NW = 32
_TRIPS = (_NCHUNK + _NW - 1) // _NW
_NPAD = 10240
_RPS = _NPAD // 16
_EB = 1280
_EGRID = E // _EB

def _sc_mesh():
    return plsc.VectorSubcoreMesh(core_axis_name="c", subcore_axis_name="s",
                                  num_cores=2, num_subcores=16)


def _sc_gather_body(hv_hbm, src_hbm, dst_hbm, outs_hbm, outd_hbm,
                    idx_s, idx_d, rows_s, rows_d, sem_s, sem_d):
    wid = lax.axis_index("s") * 2 + lax.axis_index("c")

    def body(i, carry):
        c = wid + _NW * i

        @pl.when(c < _NCHUNK)
        def _():
            base = c * _CHUNK
            pltpu.sync_copy(src_hbm.at[pl.ds(base, _CHUNK)], idx_s)
            pltpu.sync_copy(dst_hbm.at[pl.ds(base, _CHUNK)], idx_d)
            pltpu.async_copy(hv_hbm.at[idx_s], rows_s, sem_s).wait()
            pltpu.async_copy(hv_hbm.at[idx_d], rows_d, sem_d).wait()
            pltpu.sync_copy(rows_s, outs_hbm.at[pl.ds(base, _CHUNK)])
            pltpu.sync_copy(rows_d, outd_hbm.at[pl.ds(base, _CHUNK)])
        return carry

    lax.fori_loop(0, _TRIPS, body, 0)


def _sc_gather(h_v, src, dst):
    fn = pl.kernel(
        _sc_gather_body,
        out_type=(jax.ShapeDtypeStruct((E, HID), jnp.float32),
                  jax.ShapeDtypeStruct((E, HID), jnp.float32)),
        mesh=_sc_mesh(),
        scratch_types=[
            pltpu.VMEM((_CHUNK,), jnp.int32),
            pltpu.VMEM((_CHUNK,), jnp.int32),
            pltpu.VMEM((_CHUNK, HID), jnp.float32),
            pltpu.VMEM((_CHUNK, HID), jnp.float32),
            pltpu.SemaphoreType.DMA,
            pltpu.SemaphoreType.DMA,
        ],
    )
    return fn(h_v, src, dst)


def _sc_scatter_body(vals_hbm, src_hbm, z128_hbm,
                     outv0_hbm, outv1_hbm, idx_v, vals_v, accv):
    cid = lax.axis_index("c")
    sid = lax.axis_index("s")
    wid = sid * 2 + cid
    rbase = sid * _RPS
    pltpu.sync_copy(z128_hbm.at[pl.ds(0, _CHUNK)], vals_v)
    for k in range(_RPS // _CHUNK):
        pltpu.sync_copy(vals_v, accv.at[pl.ds(rbase + k * _CHUNK, _CHUNK)])
    plsc.subcore_barrier()

    def body(i, carry):
        c = wid + _NW * i

        @pl.when(c < _NCHUNK)
        def _():
            base = c * _CHUNK
            pltpu.sync_copy(src_hbm.at[pl.ds(base, _CHUNK)], idx_v)
            pltpu.sync_copy(vals_hbm.at[pl.ds(base, _CHUNK)], vals_v)
            pltpu.sync_copy(vals_v, accv.at[idx_v], add=True)
        return carry

    lax.fori_loop(0, _TRIPS, body, 0)
    plsc.subcore_barrier()

    for k in range(_RPS // _CHUNK):
        off = rbase + k * _CHUNK
        pltpu.sync_copy(accv.at[pl.ds(off, _CHUNK)], vals_v)

        @pl.when(cid == 0)
        def _():
            pltpu.sync_copy(vals_v, outv0_hbm.at[pl.ds(off, _CHUNK)])

        @pl.when(cid == 1)
        def _():
            pltpu.sync_copy(vals_v, outv1_hbm.at[pl.ds(off, _CHUNK)])


def _sc_scatter(vals, src, z128):
    fn = pl.kernel(
        _sc_scatter_body,
        out_type=(jax.ShapeDtypeStruct((_NPAD, HID), jnp.float32),
                  jax.ShapeDtypeStruct((_NPAD, HID), jnp.float32)),
        mesh=_sc_mesh(),
        scratch_types=[
            pltpu.VMEM((_CHUNK,), jnp.int32),
            pltpu.VMEM((_CHUNK, HID), jnp.float32),
            pltpu.VMEM_SHARED((_NPAD, HID), jnp.float32),
        ],
    )
    outv0, outv1 = fn(vals, src, z128)
    return jnp.stack([outv0[:N], outv1[:N]])


def _gelu(x):
    return x * 0.5 * (1.0 + lax.erf(x * (1.0 / math.sqrt(2.0))))


def _head_expand():
    col = lax.broadcasted_iota(jnp.int32, (HEADS, HID), 1) // DH
    row = lax.broadcasted_iota(jnp.int32, (HEADS, HID), 0)
    return (col == row).astype(jnp.float32)


def _edge_attn_kernel(he_ref, sc_ref, sh_ref, hs_ref, hd_ref,
                      wb1, bb1, wb2, bb2, wb3, bb3,
                      wv1, bv1, wv2, bv2, wv3, bv3,
                      evv_ref, ew_ref):
    he = he_ref[...] * sc_ref[...] + sh_ref[...]
    hs = hs_ref[...]
    hd = hd_ref[...]
    x384 = jnp.concatenate([hs, he, hd], axis=1)
    a = jnp.maximum(jnp.dot(x384, wb1[...], preferred_element_type=jnp.float32) + bb1[...], 0.0)
    a = jnp.maximum(jnp.dot(a, wb2[...], preferred_element_type=jnp.float32) + bb2[...], 0.0)
    w = jnp.dot(a, wb3[...], preferred_element_type=jnp.float32) + bb3[...]
    ew = jnp.exp(w * (1.0 / math.sqrt(DH)))
    x256 = jnp.concatenate([he, hd], axis=1)
    v = _gelu(jnp.dot(x256, wv1[...], preferred_element_type=jnp.float32) + bv1[...])
    v = _gelu(jnp.dot(v, wv2[...], preferred_element_type=jnp.float32) + bv2[...])
    v = jnp.dot(v, wv3[...], preferred_element_type=jnp.float32) + bv3[...]
    evv_ref[...] = v * jnp.dot(ew, _head_expand(), preferred_element_type=jnp.float32)
    ew_ref[...] = jnp.concatenate(
        [ew, jnp.zeros((ew.shape[0], HID - HEADS), jnp.float32)], axis=1)


def _edge_attn(he_raw, scale, shift, hs, hd, p):
    eb = lambda i: (i, 0)
    z2 = lambda i: (0, 0)
    in_specs = [
        pl.BlockSpec((_EB, HID), eb),
        pl.BlockSpec((1, HID), z2),
        pl.BlockSpec((1, HID), z2),
        pl.BlockSpec((_EB, HID), eb),
        pl.BlockSpec((_EB, HID), eb),
    ]
    weights = [p["B1"]["W"], p["B1"]["b"], p["B2"]["W"], p["B2"]["b"],
               p["B3"]["W"], p["B3"]["b"],
               p["WV1"]["W"], p["WV1"]["b"], p["WV2"]["W"], p["WV2"]["b"],
               p["WV3"]["W"], p["WV3"]["b"]]
    weights = [w if w.ndim == 2 else w.reshape(1, -1) for w in weights]
    in_specs += [pl.BlockSpec(w.shape, z2) for w in weights]
    return pl.pallas_call(
        _edge_attn_kernel,
        grid=(_EGRID,),
        in_specs=in_specs,
        out_specs=(pl.BlockSpec((_EB, HID), eb), pl.BlockSpec((_EB, HID), eb)),
        out_shape=(jax.ShapeDtypeStruct((E, HID), jnp.float32),
                   jax.ShapeDtypeStruct((E, HID), jnp.float32)),
    )(he_raw, scale, shift, hs, hd, *weights)


def _edge_mlp_kernel(he_ref, sc_ref, sh_ref, hs_ref, hd_ref,
                     we1, be1, we2, be2, we3, be3, g_ref, b_ref,
                     y_ref, sum_ref, sq_ref, osc_ref, osh_ref):
    i = pl.program_id(0)
    he = he_ref[...] * sc_ref[...] + sh_ref[...]
    x384 = jnp.concatenate([hs_ref[...], he, hd_ref[...]], axis=1)
    m = _gelu(jnp.dot(x384, we1[...], preferred_element_type=jnp.float32) + be1[...])
    m = _gelu(jnp.dot(m, we2[...], preferred_element_type=jnp.float32) + be2[...])
    m = jnp.dot(m, we3[...], preferred_element_type=jnp.float32) + be3[...]
    y = he + m
    y_ref[...] = y
    ps = jnp.sum(y, axis=0, keepdims=True)
    pq = jnp.sum(y * y, axis=0, keepdims=True)

    @pl.when(i == 0)
    def _():
        sum_ref[...] = ps
        sq_ref[...] = pq

    @pl.when(i > 0)
    def _():
        sum_ref[...] += ps
        sq_ref[...] += pq

    @pl.when(i == _EGRID - 1)
    def _():
        mean = sum_ref[...] * (1.0 / E)
        var = sq_ref[...] * (1.0 / E) - mean * mean
        sc = g_ref[...] * lax.rsqrt(var + 1e-5)
        osc_ref[...] = sc
        osh_ref[...] = b_ref[...] - mean * sc


def _edge_mlp(he_raw, scale, shift, hs, hd, p):
    eb = lambda i: (i, 0)
    z2 = lambda i: (0, 0)
    in_specs = [
        pl.BlockSpec((_EB, HID), eb),
        pl.BlockSpec((1, HID), z2),
        pl.BlockSpec((1, HID), z2),
        pl.BlockSpec((_EB, HID), eb),
        pl.BlockSpec((_EB, HID), eb),
    ]
    weights = [p["E1"]["W"], p["E1"]["b"], p["E2"]["W"], p["E2"]["b"],
               p["E3"]["W"], p["E3"]["b"], p["bne"]["g"], p["bne"]["b"]]
    weights = [w if w.ndim == 2 else w.reshape(1, -1) for w in weights]
    in_specs += [pl.BlockSpec(w.shape, z2) for w in weights]
    return pl.pallas_call(
        _edge_mlp_kernel,
        grid=(_EGRID,),
        in_specs=in_specs,
        out_specs=(pl.BlockSpec((_EB, HID), eb),
                   pl.BlockSpec((1, HID), z2), pl.BlockSpec((1, HID), z2),
                   pl.BlockSpec((1, HID), z2), pl.BlockSpec((1, HID), z2)),
        out_shape=(jax.ShapeDtypeStruct((E, HID), jnp.float32),
                   jax.ShapeDtypeStruct((1, HID), jnp.float32),
                   jax.ShapeDtypeStruct((1, HID), jnp.float32),
                   jax.ShapeDtypeStruct((1, HID), jnp.float32),
                   jax.ShapeDtypeStruct((1, HID), jnp.float32)),
    )(he_raw, scale, shift, hs, hd, *weights)


def _node_kernel(hvp_ref, denp_ref, hv_ref, bidr_ref, bidc_ref,
                 wo, g0, b0, wd1, bd1, wd2, bd2, g1, b1,
                 wg1, bg1, wg2, bg2, wg3, bg3, pre_ref, out_ref):
    hv = hvp_ref[0] + hvp_ref[1]
    den = denp_ref[0] + denp_ref[1]
    den4 = den[:, :HEADS]
    denx = jnp.dot(den4, _head_expand(), preferred_element_type=jnp.float32)
    hvn = hv / (denx + 1e-12)
    dh = jnp.dot(hvn, wo[...], preferred_element_type=jnp.float32)
    x = hv_ref[...] + dh
    m = jnp.mean(x, axis=0, keepdims=True)
    v = jnp.mean((x - m) * (x - m), axis=0, keepdims=True)
    x = (x - m) * lax.rsqrt(v + 1e-5) * g0[...] + b0[...]
    h = bd2[...] + jnp.zeros((N, HID), jnp.float32)
    for k in range(4):
        hk = jnp.maximum(
            jnp.dot(x, wd1[:, k * HID:(k + 1) * HID],
                    preferred_element_type=jnp.float32)
            + bd1[:, k * HID:(k + 1) * HID], 0.0)
        h = h + jnp.dot(hk, wd2[k * HID:(k + 1) * HID, :],
                        preferred_element_type=jnp.float32)
    x2 = x + h
    m2 = jnp.mean(x2, axis=0, keepdims=True)
    v2 = jnp.mean((x2 - m2) * (x2 - m2), axis=0, keepdims=True)
    x2 = (x2 - m2) * lax.rsqrt(v2 + 1e-5) * g1[...] + b1[...]
    pre_ref[...] = x2
    rows = lax.broadcasted_iota(jnp.int32, (NG, N), 0)
    oh = (rows == bidr_ref[...]).astype(jnp.float32)
    cnt = jnp.sum(oh, axis=1, keepdims=True)
    cv = jnp.dot(oh, x2, preferred_element_type=jnp.float32) / jnp.maximum(cnt, 1.0)
    gg = jnp.maximum(jnp.dot(cv, wg1[...], preferred_element_type=jnp.float32) + bg1[...], 0.0)
    gg = jnp.maximum(jnp.dot(gg, wg2[...], preferred_element_type=jnp.float32) + bg2[...], 0.0)
    gg = jax.nn.sigmoid(jnp.dot(gg, wg3[...], preferred_element_type=jnp.float32) + bg3[...])
    cols = lax.broadcasted_iota(jnp.int32, (N, NG), 1)
    oht = (cols == bidc_ref[...]).astype(jnp.float32)
    out_ref[...] = x2 * jnp.dot(oht, gg, preferred_element_type=jnp.float32)


def _node_update(hvp, denp, hv, bidr, bidc, p):
    weights = [p["WO"], p["bn0"]["g"], p["bn0"]["b"],
               p["D1"]["W"], p["D1"]["b"], p["D2"]["W"], p["D2"]["b"],
               p["bn1"]["g"], p["bn1"]["b"],
               p["G1"]["W"], p["G1"]["b"], p["G2"]["W"], p["G2"]["b"],
               p["G3"]["W"], p["G3"]["b"]]
    weights = [w if w.ndim == 2 else w.reshape(1, -1) for w in weights]
    return pl.pallas_call(
        _node_kernel,
        out_shape=(jax.ShapeDtypeStruct((N, HID), jnp.float32),
                   jax.ShapeDtypeStruct((N, HID), jnp.float32)),
    )(hvp, denp, hv, bidr, bidc, *weights)


def _finalize_kernel(y_ref, sc_ref, sh_ref, out_ref):
    out_ref[...] = y_ref[...] * sc_ref[...] + sh_ref[...]


def _finalize_edges(y, scale, shift):
    eb = lambda i: (i, 0)
    z2 = lambda i: (0, 0)
    return pl.pallas_call(
        _finalize_kernel,
        grid=(_EGRID,),
        in_specs=[pl.BlockSpec((_EB, HID), eb),
                  pl.BlockSpec((1, HID), z2), pl.BlockSpec((1, HID), z2)],
        out_specs=pl.BlockSpec((_EB, HID), eb),
        out_shape=jax.ShapeDtypeStruct((E, HID), jnp.float32),
    )(y, scale, shift)


def kernel(h_V, h_E, P_idx, batch_id, params):
    src = P_idx[0]
    dst = P_idx[1]
    bidr = batch_id.reshape(1, N)
    bidc = batch_id.reshape(N, 1)
    z128 = jnp.zeros((_NPAD, HID), jnp.float32)
    scale = jnp.ones((1, HID), jnp.float32)
    shift = jnp.zeros((1, HID), jnp.float32)
    he_raw = h_E
    hv = h_V
    for p in params:
        hs, hd = _sc_gather(hv, src, dst)
        evv, ewp = _edge_attn(he_raw, scale, shift, hs, hd, p)
        hvp = _sc_scatter(evv, src, z128)
        denp = _sc_scatter(ewp, src, z128)
        hv_pre, hv = _node_update(hvp, denp, hv, bidr, bidc, p)
        hs2, hd2 = _sc_gather(hv_pre, src, dst)
        he_raw, _s, _q, scale, shift = _edge_mlp(he_raw, scale, shift, hs2, hd2, p)
    he_out = _finalize_edges(he_raw, scale, shift)
    return (hv, he_out)


# speedup vs baseline: 22.2128x; 1.0700x over previous
import functools
import math

import jax
import jax.numpy as jnp
from jax import lax
from jax.experimental import pallas as pl
from jax.experimental.pallas import tpu as pltpu
from jax.experimental.pallas import tpu_sc as plsc

N = 10000
E = 320000
HID = 128
NIN = 256
HEADS = 4
DH = HID // HEADS
NG = 16

_CHUNK = 128
_SCH = 256
_NSUP = E // _SCH
_NW = 32
_STRIPS = (_NSUP + _NW - 1) // _NW
_NPAD = 10240
_RPS = _NPAD // 16
_EB = 1280
_EGRID = E // _EB

def _sc_mesh():
    return plsc.VectorSubcoreMesh(core_axis_name="c", subcore_axis_name="s",
                                  num_cores=2, num_subcores=16)


def _sc_gather_body(hv_hbm, src_hbm, dst_hbm, outs_hbm, outd_hbm,
                    idx_s0, idx_s1, idx_d0, idx_d1,
                    rows_s0, rows_s1, rows_d0, rows_d1, sem_s, sem_d):
    wid = lax.axis_index("s") * 2 + lax.axis_index("c")

    def body(i, carry):
        c = wid + _NW * i

        @pl.when(c < _NSUP)
        def _():
            base = c * _SCH
            pltpu.sync_copy(src_hbm.at[pl.ds(base, _CHUNK)], idx_s0)
            pltpu.sync_copy(src_hbm.at[pl.ds(base + _CHUNK, _CHUNK)], idx_s1)
            pltpu.sync_copy(dst_hbm.at[pl.ds(base, _CHUNK)], idx_d0)
            pltpu.sync_copy(dst_hbm.at[pl.ds(base + _CHUNK, _CHUNK)], idx_d1)
            cps = [
                pltpu.async_copy(hv_hbm.at[idx_s0], rows_s0, sem_s),
                pltpu.async_copy(hv_hbm.at[idx_s1], rows_s1, sem_s),
                pltpu.async_copy(hv_hbm.at[idx_d0], rows_d0, sem_d),
                pltpu.async_copy(hv_hbm.at[idx_d1], rows_d1, sem_d),
            ]
            for cp in cps:
                cp.wait()
            pltpu.sync_copy(rows_s0, outs_hbm.at[pl.ds(base, _CHUNK)])
            pltpu.sync_copy(rows_s1, outs_hbm.at[pl.ds(base + _CHUNK, _CHUNK)])
            pltpu.sync_copy(rows_d0, outd_hbm.at[pl.ds(base, _CHUNK)])
            pltpu.sync_copy(rows_d1, outd_hbm.at[pl.ds(base + _CHUNK, _CHUNK)])
        return carry

    lax.fori_loop(0, _STRIPS, body, 0)


def _sc_gather(h_v, src, dst):
    fn = pl.kernel(
        _sc_gather_body,
        out_type=(jax.ShapeDtypeStruct((E, HID), jnp.float32),
                  jax.ShapeDtypeStruct((E, HID), jnp.float32)),
        mesh=_sc_mesh(),
        scratch_types=[
            pltpu.VMEM((_CHUNK,), jnp.int32),
            pltpu.VMEM((_CHUNK,), jnp.int32),
            pltpu.VMEM((_CHUNK,), jnp.int32),
            pltpu.VMEM((_CHUNK,), jnp.int32),
            pltpu.VMEM((_CHUNK, HID), jnp.float32),
            pltpu.VMEM((_CHUNK, HID), jnp.float32),
            pltpu.VMEM((_CHUNK, HID), jnp.float32),
            pltpu.VMEM((_CHUNK, HID), jnp.float32),
            pltpu.SemaphoreType.DMA,
            pltpu.SemaphoreType.DMA,
        ],
    )
    return fn(h_v, src, dst)


def _sc_scatter_body(vals_hbm, src_hbm, z128_hbm,
                     outv0_hbm, outv1_hbm, idx_0, idx_1, vals_0, vals_1,
                     sem, accv):
    cid = lax.axis_index("c")
    sid = lax.axis_index("s")
    wid = sid * 2 + cid
    rbase = sid * _RPS

    if True:
        pltpu.sync_copy(z128_hbm.at[pl.ds(0, _CHUNK)], vals_0)
        for k in range(_RPS // _CHUNK):
            pltpu.sync_copy(vals_0, accv.at[pl.ds(rbase + k * _CHUNK, _CHUNK)])
        plsc.subcore_barrier()

        def body(i, carry):
            c = wid + _NW * i

            @pl.when(c < _NSUP)
            def _():
                base = c * _SCH
                pltpu.sync_copy(src_hbm.at[pl.ds(base, _CHUNK)], idx_0)
                pltpu.sync_copy(src_hbm.at[pl.ds(base + _CHUNK, _CHUNK)], idx_1)
                pltpu.sync_copy(vals_hbm.at[pl.ds(base, _CHUNK)], vals_0)
                pltpu.sync_copy(vals_hbm.at[pl.ds(base + _CHUNK, _CHUNK)], vals_1)
                cps = [
                    pltpu.async_copy(vals_0, accv.at[idx_0], sem, add=True),
                    pltpu.async_copy(vals_1, accv.at[idx_1], sem, add=True),
                ]
                for cp in cps:
                    cp.wait()
            return carry

        lax.fori_loop(0, _STRIPS, body, 0)
        plsc.subcore_barrier()

        for k in range(_RPS // _CHUNK):
            off = rbase + k * _CHUNK
            pltpu.sync_copy(accv.at[pl.ds(off, _CHUNK)], vals_0)

            @pl.when(cid == 0)
            def _():
                pltpu.sync_copy(vals_0, outv0_hbm.at[pl.ds(off, _CHUNK)])

            @pl.when(cid == 1)
            def _():
                pltpu.sync_copy(vals_0, outv1_hbm.at[pl.ds(off, _CHUNK)])


def _sc_scatter(vals, src, z128):
    fn = pl.kernel(
        _sc_scatter_body,
        out_type=(jax.ShapeDtypeStruct((_NPAD, HID), jnp.float32),
                  jax.ShapeDtypeStruct((_NPAD, HID), jnp.float32)),
        mesh=_sc_mesh(),
        scratch_types=[
            pltpu.VMEM((_CHUNK,), jnp.int32),
            pltpu.VMEM((_CHUNK,), jnp.int32),
            pltpu.VMEM((_CHUNK, HID), jnp.float32),
            pltpu.VMEM((_CHUNK, HID), jnp.float32),
            pltpu.SemaphoreType.DMA,
            pltpu.VMEM_SHARED((_NPAD, HID), jnp.float32),
        ],
    )
    outv0, outv1 = fn(vals, src, z128)
    return jnp.stack([outv0[:N], outv1[:N]])


def _gelu(x):
    return x * 0.5 * (1.0 + lax.erf(x * (1.0 / math.sqrt(2.0))))


def _head_expand():
    col = lax.broadcasted_iota(jnp.int32, (HEADS, HID), 1) // DH
    row = lax.broadcasted_iota(jnp.int32, (HEADS, HID), 0)
    return (col == row).astype(jnp.float32)


def _edge_attn_kernel(he_ref, sc_ref, sh_ref, hs_ref, hd_ref,
                      wb1, bb1, wb2, bb2, wb3, bb3,
                      wv1, bv1, wv2, bv2, wv3, bv3,
                      evv_ref, ew_ref):
    he = he_ref[...] * sc_ref[...] + sh_ref[...]
    hs = hs_ref[...]
    hd = hd_ref[...]
    x384 = jnp.concatenate([hs, he, hd], axis=1)
    a = jnp.maximum(jnp.dot(x384, wb1[...], preferred_element_type=jnp.float32) + bb1[...], 0.0)
    a = jnp.maximum(jnp.dot(a, wb2[...], preferred_element_type=jnp.float32) + bb2[...], 0.0)
    w = jnp.dot(a, wb3[...], preferred_element_type=jnp.float32) + bb3[...]
    ew = jnp.exp(w * (1.0 / math.sqrt(DH)))
    x256 = jnp.concatenate([he, hd], axis=1)
    v = _gelu(jnp.dot(x256, wv1[...], preferred_element_type=jnp.float32) + bv1[...])
    v = _gelu(jnp.dot(v, wv2[...], preferred_element_type=jnp.float32) + bv2[...])
    v = jnp.dot(v, wv3[...], preferred_element_type=jnp.float32) + bv3[...]
    evv_ref[...] = v * jnp.dot(ew, _head_expand(), preferred_element_type=jnp.float32)
    ew_ref[...] = jnp.concatenate(
        [ew, jnp.zeros((ew.shape[0], HID - HEADS), jnp.float32)], axis=1)


def _edge_attn(he_raw, scale, shift, hs, hd, p):
    eb = lambda i: (i, 0)
    z2 = lambda i: (0, 0)
    in_specs = [
        pl.BlockSpec((_EB, HID), eb),
        pl.BlockSpec((1, HID), z2),
        pl.BlockSpec((1, HID), z2),
        pl.BlockSpec((_EB, HID), eb),
        pl.BlockSpec((_EB, HID), eb),
    ]
    weights = [p["B1"]["W"], p["B1"]["b"], p["B2"]["W"], p["B2"]["b"],
               p["B3"]["W"], p["B3"]["b"],
               p["WV1"]["W"], p["WV1"]["b"], p["WV2"]["W"], p["WV2"]["b"],
               p["WV3"]["W"], p["WV3"]["b"]]
    weights = [w if w.ndim == 2 else w.reshape(1, -1) for w in weights]
    in_specs += [pl.BlockSpec(w.shape, z2) for w in weights]
    return pl.pallas_call(
        _edge_attn_kernel,
        grid=(_EGRID,),
        in_specs=in_specs,
        out_specs=(pl.BlockSpec((_EB, HID), eb), pl.BlockSpec((_EB, HID), eb)),
        out_shape=(jax.ShapeDtypeStruct((E, HID), jnp.float32),
                   jax.ShapeDtypeStruct((E, HID), jnp.float32)),
    )(he_raw, scale, shift, hs, hd, *weights)


def _edge_mlp_kernel(he_ref, sc_ref, sh_ref, hs_ref, hd_ref,
                     we1, be1, we2, be2, we3, be3, g_ref, b_ref,
                     y_ref, sum_ref, sq_ref, osc_ref, osh_ref):
    i = pl.program_id(0)
    he = he_ref[...] * sc_ref[...] + sh_ref[...]
    x384 = jnp.concatenate([hs_ref[...], he, hd_ref[...]], axis=1)
    m = _gelu(jnp.dot(x384, we1[...], preferred_element_type=jnp.float32) + be1[...])
    m = _gelu(jnp.dot(m, we2[...], preferred_element_type=jnp.float32) + be2[...])
    m = jnp.dot(m, we3[...], preferred_element_type=jnp.float32) + be3[...]
    y = he + m
    y_ref[...] = y
    ps = jnp.sum(y, axis=0, keepdims=True)
    pq = jnp.sum(y * y, axis=0, keepdims=True)

    @pl.when(i == 0)
    def _():
        sum_ref[...] = ps
        sq_ref[...] = pq

    @pl.when(i > 0)
    def _():
        sum_ref[...] += ps
        sq_ref[...] += pq

    @pl.when(i == _EGRID - 1)
    def _():
        mean = sum_ref[...] * (1.0 / E)
        var = sq_ref[...] * (1.0 / E) - mean * mean
        sc = g_ref[...] * lax.rsqrt(var + 1e-5)
        osc_ref[...] = sc
        osh_ref[...] = b_ref[...] - mean * sc


def _edge_mlp(he_raw, scale, shift, hs, hd, p):
    eb = lambda i: (i, 0)
    z2 = lambda i: (0, 0)
    in_specs = [
        pl.BlockSpec((_EB, HID), eb),
        pl.BlockSpec((1, HID), z2),
        pl.BlockSpec((1, HID), z2),
        pl.BlockSpec((_EB, HID), eb),
        pl.BlockSpec((_EB, HID), eb),
    ]
    weights = [p["E1"]["W"], p["E1"]["b"], p["E2"]["W"], p["E2"]["b"],
               p["E3"]["W"], p["E3"]["b"], p["bne"]["g"], p["bne"]["b"]]
    weights = [w if w.ndim == 2 else w.reshape(1, -1) for w in weights]
    in_specs += [pl.BlockSpec(w.shape, z2) for w in weights]
    return pl.pallas_call(
        _edge_mlp_kernel,
        grid=(_EGRID,),
        in_specs=in_specs,
        out_specs=(pl.BlockSpec((_EB, HID), eb),
                   pl.BlockSpec((1, HID), z2), pl.BlockSpec((1, HID), z2),
                   pl.BlockSpec((1, HID), z2), pl.BlockSpec((1, HID), z2)),
        out_shape=(jax.ShapeDtypeStruct((E, HID), jnp.float32),
                   jax.ShapeDtypeStruct((1, HID), jnp.float32),
                   jax.ShapeDtypeStruct((1, HID), jnp.float32),
                   jax.ShapeDtypeStruct((1, HID), jnp.float32),
                   jax.ShapeDtypeStruct((1, HID), jnp.float32)),
    )(he_raw, scale, shift, hs, hd, *weights)


def _node_kernel(hvp_ref, denp_ref, hv_ref, bidr_ref, bidc_ref,
                 wo, g0, b0, wd1, bd1, wd2, bd2, g1, b1,
                 wg1, bg1, wg2, bg2, wg3, bg3, pre_ref, out_ref):
    hv = hvp_ref[0] + hvp_ref[1]
    den = denp_ref[0] + denp_ref[1]
    den4 = den[:, :HEADS]
    denx = jnp.dot(den4, _head_expand(), preferred_element_type=jnp.float32)
    hvn = hv / (denx + 1e-12)
    dh = jnp.dot(hvn, wo[...], preferred_element_type=jnp.float32)
    x = hv_ref[...] + dh
    m = jnp.mean(x, axis=0, keepdims=True)
    v = jnp.mean((x - m) * (x - m), axis=0, keepdims=True)
    x = (x - m) * lax.rsqrt(v + 1e-5) * g0[...] + b0[...]
    h = bd2[...] + jnp.zeros((N, HID), jnp.float32)
    for k in range(4):
        hk = jnp.maximum(
            jnp.dot(x, wd1[:, k * HID:(k + 1) * HID],
                    preferred_element_type=jnp.float32)
            + bd1[:, k * HID:(k + 1) * HID], 0.0)
        h = h + jnp.dot(hk, wd2[k * HID:(k + 1) * HID, :],
                        preferred_element_type=jnp.float32)
    x2 = x + h
    m2 = jnp.mean(x2, axis=0, keepdims=True)
    v2 = jnp.mean((x2 - m2) * (x2 - m2), axis=0, keepdims=True)
    x2 = (x2 - m2) * lax.rsqrt(v2 + 1e-5) * g1[...] + b1[...]
    pre_ref[...] = x2
    rows = lax.broadcasted_iota(jnp.int32, (NG, N), 0)
    oh = (rows == bidr_ref[...]).astype(jnp.float32)
    cnt = jnp.sum(oh, axis=1, keepdims=True)
    cv = jnp.dot(oh, x2, preferred_element_type=jnp.float32) / jnp.maximum(cnt, 1.0)
    gg = jnp.maximum(jnp.dot(cv, wg1[...], preferred_element_type=jnp.float32) + bg1[...], 0.0)
    gg = jnp.maximum(jnp.dot(gg, wg2[...], preferred_element_type=jnp.float32) + bg2[...], 0.0)
    gg = jax.nn.sigmoid(jnp.dot(gg, wg3[...], preferred_element_type=jnp.float32) + bg3[...])
    cols = lax.broadcasted_iota(jnp.int32, (N, NG), 1)
    oht = (cols == bidc_ref[...]).astype(jnp.float32)
    out_ref[...] = x2 * jnp.dot(oht, gg, preferred_element_type=jnp.float32)


def _node_update(hvp, denp, hv, bidr, bidc, p):
    weights = [p["WO"], p["bn0"]["g"], p["bn0"]["b"],
               p["D1"]["W"], p["D1"]["b"], p["D2"]["W"], p["D2"]["b"],
               p["bn1"]["g"], p["bn1"]["b"],
               p["G1"]["W"], p["G1"]["b"], p["G2"]["W"], p["G2"]["b"],
               p["G3"]["W"], p["G3"]["b"]]
    weights = [w if w.ndim == 2 else w.reshape(1, -1) for w in weights]
    return pl.pallas_call(
        _node_kernel,
        out_shape=(jax.ShapeDtypeStruct((N, HID), jnp.float32),
                   jax.ShapeDtypeStruct((N, HID), jnp.float32)),
    )(hvp, denp, hv, bidr, bidc, *weights)


def _finalize_kernel(y_ref, sc_ref, sh_ref, out_ref):
    out_ref[...] = y_ref[...] * sc_ref[...] + sh_ref[...]


def _finalize_edges(y, scale, shift):
    eb = lambda i: (i, 0)
    z2 = lambda i: (0, 0)
    return pl.pallas_call(
        _finalize_kernel,
        grid=(_EGRID,),
        in_specs=[pl.BlockSpec((_EB, HID), eb),
                  pl.BlockSpec((1, HID), z2), pl.BlockSpec((1, HID), z2)],
        out_specs=pl.BlockSpec((_EB, HID), eb),
        out_shape=jax.ShapeDtypeStruct((E, HID), jnp.float32),
    )(y, scale, shift)


def kernel(h_V, h_E, P_idx, batch_id, params):
    src = P_idx[0]
    dst = P_idx[1]
    bidr = batch_id.reshape(1, N)
    bidc = batch_id.reshape(N, 1)
    z128 = jnp.zeros((_NPAD, HID), jnp.float32)
    scale = jnp.ones((1, HID), jnp.float32)
    shift = jnp.zeros((1, HID), jnp.float32)
    he_raw = h_E
    hv = h_V
    for p in params:
        hs, hd = _sc_gather(hv, src, dst)
        evv, ewp = _edge_attn(he_raw, scale, shift, hs, hd, p)
        hvp = _sc_scatter(evv, src, z128)
        denp = _sc_scatter(ewp, src, z128)
        hv_pre, hv = _node_update(hvp, denp, hv, bidr, bidc, p)
        hs2, hd2 = _sc_gather(hv_pre, src, dst)
        he_raw, _s, _q, scale, shift = _edge_mlp(he_raw, scale, shift, hs2, hd2, p)
    he_out = _finalize_edges(he_raw, scale, shift)
    return (hv, he_out)


# speedup vs baseline: 23.5172x; 1.0587x over previous
import functools
import math

import jax
import jax.numpy as jnp
from jax import lax
from jax.experimental import pallas as pl
from jax.experimental.pallas import tpu as pltpu
from jax.experimental.pallas import tpu_sc as plsc

N = 10000
E = 320000
HID = 128
NIN = 256
HEADS = 4
DH = HID // HEADS
NG = 16

_CHUNK = 128
_SCH = 256
_NSUP = E // _SCH
_NW = 32
_STRIPS = (_NSUP + _NW - 1) // _NW
_NPAD = 10240
_RPS = _NPAD // 16
_EB = 1280
_EGRID = E // _EB

def _sc_mesh():
    return plsc.VectorSubcoreMesh(core_axis_name="c", subcore_axis_name="s",
                                  num_cores=2, num_subcores=16)


def _sc_gather_body(hv_hbm, src_hbm, dst_hbm, outs_hbm, outd_hbm,
                    idx_s0, idx_s1, idx_d0, idx_d1,
                    rows_s0, rows_s1, rows_d0, rows_d1,
                    sem_i, sem_g, sem_w):
    wid = lax.axis_index("s") * 2 + lax.axis_index("c")
    rows = (rows_s0, rows_s1, rows_d0, rows_d1)

    def body(i, carry):
        c = wid + _NW * i

        @pl.when(c < _NSUP)
        def _():
            base = c * _SCH
            cis = [
                pltpu.async_copy(src_hbm.at[pl.ds(base, _CHUNK)], idx_s0, sem_i),
                pltpu.async_copy(src_hbm.at[pl.ds(base + _CHUNK, _CHUNK)], idx_s1, sem_i),
                pltpu.async_copy(dst_hbm.at[pl.ds(base, _CHUNK)], idx_d0, sem_i),
                pltpu.async_copy(dst_hbm.at[pl.ds(base + _CHUNK, _CHUNK)], idx_d1, sem_i),
            ]

            @pl.when(i > 0)
            def _():
                for r in rows:
                    pltpu.make_async_copy(hv_hbm.at[pl.ds(0, _CHUNK)], r, sem_w).wait()

            for cp in cis:
                cp.wait()
            cps = [
                pltpu.async_copy(hv_hbm.at[idx_s0], rows_s0, sem_g),
                pltpu.async_copy(hv_hbm.at[idx_s1], rows_s1, sem_g),
                pltpu.async_copy(hv_hbm.at[idx_d0], rows_d0, sem_g),
                pltpu.async_copy(hv_hbm.at[idx_d1], rows_d1, sem_g),
            ]
            for cp in cps:
                cp.wait()
            pltpu.async_copy(rows_s0, outs_hbm.at[pl.ds(base, _CHUNK)], sem_w)
            pltpu.async_copy(rows_s1, outs_hbm.at[pl.ds(base + _CHUNK, _CHUNK)], sem_w)
            pltpu.async_copy(rows_d0, outd_hbm.at[pl.ds(base, _CHUNK)], sem_w)
            pltpu.async_copy(rows_d1, outd_hbm.at[pl.ds(base + _CHUNK, _CHUNK)], sem_w)
        return carry

    lax.fori_loop(0, _STRIPS, body, 0)
    for r in rows:
        pltpu.make_async_copy(hv_hbm.at[pl.ds(0, _CHUNK)], r, sem_w).wait()


def _sc_gather(h_v, src, dst):
    fn = pl.kernel(
        _sc_gather_body,
        out_type=(jax.ShapeDtypeStruct((E, HID), jnp.float32),
                  jax.ShapeDtypeStruct((E, HID), jnp.float32)),
        mesh=_sc_mesh(),
        scratch_types=[
            pltpu.VMEM((_CHUNK,), jnp.int32),
            pltpu.VMEM((_CHUNK,), jnp.int32),
            pltpu.VMEM((_CHUNK,), jnp.int32),
            pltpu.VMEM((_CHUNK,), jnp.int32),
            pltpu.VMEM((_CHUNK, HID), jnp.float32),
            pltpu.VMEM((_CHUNK, HID), jnp.float32),
            pltpu.VMEM((_CHUNK, HID), jnp.float32),
            pltpu.VMEM((_CHUNK, HID), jnp.float32),
            pltpu.SemaphoreType.DMA,
            pltpu.SemaphoreType.DMA,
            pltpu.SemaphoreType.DMA,
        ],
    )
    return fn(h_v, src, dst)


def _sc_scatter_body(vals_hbm, src_hbm, z128_hbm,
                     outv0_hbm, outv1_hbm, idx_0, idx_1, vals_0, vals_1,
                     sem, accv):
    cid = lax.axis_index("c")
    sid = lax.axis_index("s")
    wid = sid * 2 + cid
    rbase = sid * _RPS

    if True:
        pltpu.sync_copy(z128_hbm.at[pl.ds(0, _CHUNK)], vals_0)
        for k in range(_RPS // _CHUNK):
            pltpu.sync_copy(vals_0, accv.at[pl.ds(rbase + k * _CHUNK, _CHUNK)])
        plsc.subcore_barrier()

        def body(i, carry):
            c = wid + _NW * i

            @pl.when(c < _NSUP)
            def _():
                base = c * _SCH
                pltpu.sync_copy(src_hbm.at[pl.ds(base, _CHUNK)], idx_0)
                pltpu.sync_copy(src_hbm.at[pl.ds(base + _CHUNK, _CHUNK)], idx_1)
                pltpu.sync_copy(vals_hbm.at[pl.ds(base, _CHUNK)], vals_0)
                pltpu.sync_copy(vals_hbm.at[pl.ds(base + _CHUNK, _CHUNK)], vals_1)
                cps = [
                    pltpu.async_copy(vals_0, accv.at[idx_0], sem, add=True),
                    pltpu.async_copy(vals_1, accv.at[idx_1], sem, add=True),
                ]
                for cp in cps:
                    cp.wait()
            return carry

        lax.fori_loop(0, _STRIPS, body, 0)
        plsc.subcore_barrier()

        for k in range(_RPS // _CHUNK):
            off = rbase + k * _CHUNK
            pltpu.sync_copy(accv.at[pl.ds(off, _CHUNK)], vals_0)

            @pl.when(cid == 0)
            def _():
                pltpu.sync_copy(vals_0, outv0_hbm.at[pl.ds(off, _CHUNK)])

            @pl.when(cid == 1)
            def _():
                pltpu.sync_copy(vals_0, outv1_hbm.at[pl.ds(off, _CHUNK)])


def _sc_scatter(vals, src, z128):
    fn = pl.kernel(
        _sc_scatter_body,
        out_type=(jax.ShapeDtypeStruct((_NPAD, HID), jnp.float32),
                  jax.ShapeDtypeStruct((_NPAD, HID), jnp.float32)),
        mesh=_sc_mesh(),
        scratch_types=[
            pltpu.VMEM((_CHUNK,), jnp.int32),
            pltpu.VMEM((_CHUNK,), jnp.int32),
            pltpu.VMEM((_CHUNK, HID), jnp.float32),
            pltpu.VMEM((_CHUNK, HID), jnp.float32),
            pltpu.SemaphoreType.DMA,
            pltpu.VMEM_SHARED((_NPAD, HID), jnp.float32),
        ],
    )
    outv0, outv1 = fn(vals, src, z128)
    return jnp.stack([outv0[:N], outv1[:N]])


def _gelu(x):
    return x * 0.5 * (1.0 + lax.erf(x * (1.0 / math.sqrt(2.0))))


def _head_expand():
    col = lax.broadcasted_iota(jnp.int32, (HEADS, HID), 1) // DH
    row = lax.broadcasted_iota(jnp.int32, (HEADS, HID), 0)
    return (col == row).astype(jnp.float32)


def _edge_attn_kernel(he_ref, sc_ref, sh_ref, hs_ref, hd_ref,
                      wb1, bb1, wb2, bb2, wb3, bb3,
                      wv1, bv1, wv2, bv2, wv3, bv3,
                      evv_ref, ew_ref):
    he = he_ref[...] * sc_ref[...] + sh_ref[...]
    hs = hs_ref[...]
    hd = hd_ref[...]
    x384 = jnp.concatenate([hs, he, hd], axis=1)
    a = jnp.maximum(jnp.dot(x384, wb1[...], preferred_element_type=jnp.float32) + bb1[...], 0.0)
    a = jnp.maximum(jnp.dot(a, wb2[...], preferred_element_type=jnp.float32) + bb2[...], 0.0)
    w = jnp.dot(a, wb3[...], preferred_element_type=jnp.float32) + bb3[...]
    ew = jnp.exp(w * (1.0 / math.sqrt(DH)))
    x256 = jnp.concatenate([he, hd], axis=1)
    v = _gelu(jnp.dot(x256, wv1[...], preferred_element_type=jnp.float32) + bv1[...])
    v = _gelu(jnp.dot(v, wv2[...], preferred_element_type=jnp.float32) + bv2[...])
    v = jnp.dot(v, wv3[...], preferred_element_type=jnp.float32) + bv3[...]
    evv_ref[...] = v * jnp.dot(ew, _head_expand(), preferred_element_type=jnp.float32)
    ew_ref[...] = jnp.concatenate(
        [ew, jnp.zeros((ew.shape[0], HID - HEADS), jnp.float32)], axis=1)


def _edge_attn(he_raw, scale, shift, hs, hd, p):
    eb = lambda i: (i, 0)
    z2 = lambda i: (0, 0)
    in_specs = [
        pl.BlockSpec((_EB, HID), eb),
        pl.BlockSpec((1, HID), z2),
        pl.BlockSpec((1, HID), z2),
        pl.BlockSpec((_EB, HID), eb),
        pl.BlockSpec((_EB, HID), eb),
    ]
    weights = [p["B1"]["W"], p["B1"]["b"], p["B2"]["W"], p["B2"]["b"],
               p["B3"]["W"], p["B3"]["b"],
               p["WV1"]["W"], p["WV1"]["b"], p["WV2"]["W"], p["WV2"]["b"],
               p["WV3"]["W"], p["WV3"]["b"]]
    weights = [w if w.ndim == 2 else w.reshape(1, -1) for w in weights]
    in_specs += [pl.BlockSpec(w.shape, z2) for w in weights]
    return pl.pallas_call(
        _edge_attn_kernel,
        grid=(_EGRID,),
        in_specs=in_specs,
        out_specs=(pl.BlockSpec((_EB, HID), eb), pl.BlockSpec((_EB, HID), eb)),
        out_shape=(jax.ShapeDtypeStruct((E, HID), jnp.float32),
                   jax.ShapeDtypeStruct((E, HID), jnp.float32)),
    )(he_raw, scale, shift, hs, hd, *weights)


def _edge_mlp_kernel(he_ref, sc_ref, sh_ref, hs_ref, hd_ref,
                     we1, be1, we2, be2, we3, be3, g_ref, b_ref,
                     y_ref, sum_ref, sq_ref, osc_ref, osh_ref):
    i = pl.program_id(0)
    he = he_ref[...] * sc_ref[...] + sh_ref[...]
    x384 = jnp.concatenate([hs_ref[...], he, hd_ref[...]], axis=1)
    m = _gelu(jnp.dot(x384, we1[...], preferred_element_type=jnp.float32) + be1[...])
    m = _gelu(jnp.dot(m, we2[...], preferred_element_type=jnp.float32) + be2[...])
    m = jnp.dot(m, we3[...], preferred_element_type=jnp.float32) + be3[...]
    y = he + m
    y_ref[...] = y
    ps = jnp.sum(y, axis=0, keepdims=True)
    pq = jnp.sum(y * y, axis=0, keepdims=True)

    @pl.when(i == 0)
    def _():
        sum_ref[...] = ps
        sq_ref[...] = pq

    @pl.when(i > 0)
    def _():
        sum_ref[...] += ps
        sq_ref[...] += pq

    @pl.when(i == _EGRID - 1)
    def _():
        mean = sum_ref[...] * (1.0 / E)
        var = sq_ref[...] * (1.0 / E) - mean * mean
        sc = g_ref[...] * lax.rsqrt(var + 1e-5)
        osc_ref[...] = sc
        osh_ref[...] = b_ref[...] - mean * sc


def _edge_mlp(he_raw, scale, shift, hs, hd, p):
    eb = lambda i: (i, 0)
    z2 = lambda i: (0, 0)
    in_specs = [
        pl.BlockSpec((_EB, HID), eb),
        pl.BlockSpec((1, HID), z2),
        pl.BlockSpec((1, HID), z2),
        pl.BlockSpec((_EB, HID), eb),
        pl.BlockSpec((_EB, HID), eb),
    ]
    weights = [p["E1"]["W"], p["E1"]["b"], p["E2"]["W"], p["E2"]["b"],
               p["E3"]["W"], p["E3"]["b"], p["bne"]["g"], p["bne"]["b"]]
    weights = [w if w.ndim == 2 else w.reshape(1, -1) for w in weights]
    in_specs += [pl.BlockSpec(w.shape, z2) for w in weights]
    return pl.pallas_call(
        _edge_mlp_kernel,
        grid=(_EGRID,),
        in_specs=in_specs,
        out_specs=(pl.BlockSpec((_EB, HID), eb),
                   pl.BlockSpec((1, HID), z2), pl.BlockSpec((1, HID), z2),
                   pl.BlockSpec((1, HID), z2), pl.BlockSpec((1, HID), z2)),
        out_shape=(jax.ShapeDtypeStruct((E, HID), jnp.float32),
                   jax.ShapeDtypeStruct((1, HID), jnp.float32),
                   jax.ShapeDtypeStruct((1, HID), jnp.float32),
                   jax.ShapeDtypeStruct((1, HID), jnp.float32),
                   jax.ShapeDtypeStruct((1, HID), jnp.float32)),
    )(he_raw, scale, shift, hs, hd, *weights)


def _node_kernel(hvp_ref, denp_ref, hv_ref, bidr_ref, bidc_ref,
                 wo, g0, b0, wd1, bd1, wd2, bd2, g1, b1,
                 wg1, bg1, wg2, bg2, wg3, bg3, pre_ref, out_ref):
    hv = hvp_ref[0] + hvp_ref[1]
    den = denp_ref[0] + denp_ref[1]
    den4 = den[:, :HEADS]
    denx = jnp.dot(den4, _head_expand(), preferred_element_type=jnp.float32)
    hvn = hv / (denx + 1e-12)
    dh = jnp.dot(hvn, wo[...], preferred_element_type=jnp.float32)
    x = hv_ref[...] + dh
    m = jnp.mean(x, axis=0, keepdims=True)
    v = jnp.mean((x - m) * (x - m), axis=0, keepdims=True)
    x = (x - m) * lax.rsqrt(v + 1e-5) * g0[...] + b0[...]
    h = bd2[...] + jnp.zeros((N, HID), jnp.float32)
    for k in range(4):
        hk = jnp.maximum(
            jnp.dot(x, wd1[:, k * HID:(k + 1) * HID],
                    preferred_element_type=jnp.float32)
            + bd1[:, k * HID:(k + 1) * HID], 0.0)
        h = h + jnp.dot(hk, wd2[k * HID:(k + 1) * HID, :],
                        preferred_element_type=jnp.float32)
    x2 = x + h
    m2 = jnp.mean(x2, axis=0, keepdims=True)
    v2 = jnp.mean((x2 - m2) * (x2 - m2), axis=0, keepdims=True)
    x2 = (x2 - m2) * lax.rsqrt(v2 + 1e-5) * g1[...] + b1[...]
    pre_ref[...] = x2
    rows = lax.broadcasted_iota(jnp.int32, (NG, N), 0)
    oh = (rows == bidr_ref[...]).astype(jnp.float32)
    cnt = jnp.sum(oh, axis=1, keepdims=True)
    cv = jnp.dot(oh, x2, preferred_element_type=jnp.float32) / jnp.maximum(cnt, 1.0)
    gg = jnp.maximum(jnp.dot(cv, wg1[...], preferred_element_type=jnp.float32) + bg1[...], 0.0)
    gg = jnp.maximum(jnp.dot(gg, wg2[...], preferred_element_type=jnp.float32) + bg2[...], 0.0)
    gg = jax.nn.sigmoid(jnp.dot(gg, wg3[...], preferred_element_type=jnp.float32) + bg3[...])
    cols = lax.broadcasted_iota(jnp.int32, (N, NG), 1)
    oht = (cols == bidc_ref[...]).astype(jnp.float32)
    out_ref[...] = x2 * jnp.dot(oht, gg, preferred_element_type=jnp.float32)


def _node_update(hvp, denp, hv, bidr, bidc, p):
    weights = [p["WO"], p["bn0"]["g"], p["bn0"]["b"],
               p["D1"]["W"], p["D1"]["b"], p["D2"]["W"], p["D2"]["b"],
               p["bn1"]["g"], p["bn1"]["b"],
               p["G1"]["W"], p["G1"]["b"], p["G2"]["W"], p["G2"]["b"],
               p["G3"]["W"], p["G3"]["b"]]
    weights = [w if w.ndim == 2 else w.reshape(1, -1) for w in weights]
    return pl.pallas_call(
        _node_kernel,
        out_shape=(jax.ShapeDtypeStruct((N, HID), jnp.float32),
                   jax.ShapeDtypeStruct((N, HID), jnp.float32)),
    )(hvp, denp, hv, bidr, bidc, *weights)


def _finalize_kernel(y_ref, sc_ref, sh_ref, out_ref):
    out_ref[...] = y_ref[...] * sc_ref[...] + sh_ref[...]


def _finalize_edges(y, scale, shift):
    eb = lambda i: (i, 0)
    z2 = lambda i: (0, 0)
    return pl.pallas_call(
        _finalize_kernel,
        grid=(_EGRID,),
        in_specs=[pl.BlockSpec((_EB, HID), eb),
                  pl.BlockSpec((1, HID), z2), pl.BlockSpec((1, HID), z2)],
        out_specs=pl.BlockSpec((_EB, HID), eb),
        out_shape=jax.ShapeDtypeStruct((E, HID), jnp.float32),
    )(y, scale, shift)


def kernel(h_V, h_E, P_idx, batch_id, params):
    src = P_idx[0]
    dst = P_idx[1]
    bidr = batch_id.reshape(1, N)
    bidc = batch_id.reshape(N, 1)
    z128 = jnp.zeros((_NPAD, HID), jnp.float32)
    scale = jnp.ones((1, HID), jnp.float32)
    shift = jnp.zeros((1, HID), jnp.float32)
    he_raw = h_E
    hv = h_V
    for p in params:
        hs, hd = _sc_gather(hv, src, dst)
        evv, ewp = _edge_attn(he_raw, scale, shift, hs, hd, p)
        hvp = _sc_scatter(evv, src, z128)
        denp = _sc_scatter(ewp, src, z128)
        hv_pre, hv = _node_update(hvp, denp, hv, bidr, bidc, p)
        hs2, hd2 = _sc_gather(hv_pre, src, dst)
        he_raw, _s, _q, scale, shift = _edge_mlp(he_raw, scale, shift, hs2, hd2, p)
    he_out = _finalize_edges(he_raw, scale, shift)
    return (hv, he_out)


# speedup vs baseline: 25.2417x; 1.0733x over previous
import functools
import math

import jax
import jax.numpy as jnp
from jax import lax
from jax.experimental import pallas as pl
from jax.experimental.pallas import tpu as pltpu
from jax.experimental.pallas import tpu_sc as plsc

N = 10000
E = 320000
HID = 128
NIN = 256
HEADS = 4
DH = HID // HEADS
NG = 16

_CHUNK = 128
_SCH = 256
_NSUP = E // _SCH
_NW = 32
_STRIPS = (_NSUP + _NW - 1) // _NW
_NPAD = 10240
_RPS = _NPAD // 16
_EB = 1280
_EGRID = E // _EB

def _sc_mesh():
    return plsc.VectorSubcoreMesh(core_axis_name="c", subcore_axis_name="s",
                                  num_cores=2, num_subcores=16)


def _sc_gather_body(hv_hbm, src_hbm, dst_hbm, outs_hbm, outd_hbm,
                    idx_s0, idx_s1, idx_d0, idx_d1,
                    rows_s0, rows_s1, rows_d0, rows_d1,
                    sem_i, sem_g, sem_w):
    wid = lax.axis_index("s") * 2 + lax.axis_index("c")
    rows = (rows_s0, rows_s1, rows_d0, rows_d1)

    def body(i, carry):
        c = wid + _NW * i

        @pl.when(c < _NSUP)
        def _():
            base = c * _SCH
            cis = [
                pltpu.async_copy(src_hbm.at[pl.ds(base, _CHUNK)], idx_s0, sem_i),
                pltpu.async_copy(src_hbm.at[pl.ds(base + _CHUNK, _CHUNK)], idx_s1, sem_i),
                pltpu.async_copy(dst_hbm.at[pl.ds(base, _CHUNK)], idx_d0, sem_i),
                pltpu.async_copy(dst_hbm.at[pl.ds(base + _CHUNK, _CHUNK)], idx_d1, sem_i),
            ]

            @pl.when(i > 0)
            def _():
                for r in rows:
                    pltpu.make_async_copy(hv_hbm.at[pl.ds(0, _CHUNK)], r, sem_w).wait()

            for cp in cis:
                cp.wait()
            cps = [
                pltpu.async_copy(hv_hbm.at[idx_s0], rows_s0, sem_g),
                pltpu.async_copy(hv_hbm.at[idx_s1], rows_s1, sem_g),
                pltpu.async_copy(hv_hbm.at[idx_d0], rows_d0, sem_g),
                pltpu.async_copy(hv_hbm.at[idx_d1], rows_d1, sem_g),
            ]
            for cp in cps:
                cp.wait()
            pltpu.async_copy(rows_s0, outs_hbm.at[pl.ds(base, _CHUNK)], sem_w)
            pltpu.async_copy(rows_s1, outs_hbm.at[pl.ds(base + _CHUNK, _CHUNK)], sem_w)
            pltpu.async_copy(rows_d0, outd_hbm.at[pl.ds(base, _CHUNK)], sem_w)
            pltpu.async_copy(rows_d1, outd_hbm.at[pl.ds(base + _CHUNK, _CHUNK)], sem_w)
        return carry

    lax.fori_loop(0, _STRIPS, body, 0)
    for r in rows:
        pltpu.make_async_copy(hv_hbm.at[pl.ds(0, _CHUNK)], r, sem_w).wait()


def _sc_gather(h_v, src, dst):
    fn = pl.kernel(
        _sc_gather_body,
        out_type=(jax.ShapeDtypeStruct((E, HID), jnp.float32),
                  jax.ShapeDtypeStruct((E, HID), jnp.float32)),
        mesh=_sc_mesh(),
        scratch_types=[
            pltpu.VMEM((_CHUNK,), jnp.int32),
            pltpu.VMEM((_CHUNK,), jnp.int32),
            pltpu.VMEM((_CHUNK,), jnp.int32),
            pltpu.VMEM((_CHUNK,), jnp.int32),
            pltpu.VMEM((_CHUNK, HID), jnp.float32),
            pltpu.VMEM((_CHUNK, HID), jnp.float32),
            pltpu.VMEM((_CHUNK, HID), jnp.float32),
            pltpu.VMEM((_CHUNK, HID), jnp.float32),
            pltpu.SemaphoreType.DMA,
            pltpu.SemaphoreType.DMA,
            pltpu.SemaphoreType.DMA,
        ],
    )
    return fn(h_v, src, dst)


def _sc_scatter_body(vals_hbm, src_hbm, z128_hbm,
                     outv0_hbm, outv1_hbm, idx_0, idx_1, vals_0, vals_1,
                     sem_l, sem, accv):
    cid = lax.axis_index("c")
    sid = lax.axis_index("s")
    wid = sid * 2 + cid
    rbase = sid * _RPS

    if True:
        pltpu.sync_copy(z128_hbm.at[pl.ds(0, _CHUNK)], vals_0)
        for k in range(_RPS // _CHUNK):
            pltpu.sync_copy(vals_0, accv.at[pl.ds(rbase + k * _CHUNK, _CHUNK)])
        plsc.subcore_barrier()

        def body(i, carry):
            c = wid + _NW * i

            @pl.when(c < _NSUP)
            def _():
                base = c * _SCH
                cls = [
                    pltpu.async_copy(src_hbm.at[pl.ds(base, _CHUNK)], idx_0, sem_l),
                    pltpu.async_copy(src_hbm.at[pl.ds(base + _CHUNK, _CHUNK)], idx_1, sem_l),
                    pltpu.async_copy(vals_hbm.at[pl.ds(base, _CHUNK)], vals_0, sem_l),
                    pltpu.async_copy(vals_hbm.at[pl.ds(base + _CHUNK, _CHUNK)], vals_1, sem_l),
                ]
                for cp in cls:
                    cp.wait()
                cps = [
                    pltpu.async_copy(vals_0, accv.at[idx_0], sem, add=True),
                    pltpu.async_copy(vals_1, accv.at[idx_1], sem, add=True),
                ]
                for cp in cps:
                    cp.wait()
            return carry

        lax.fori_loop(0, _STRIPS, body, 0)
        plsc.subcore_barrier()

        for k in range(_RPS // _CHUNK):
            off = rbase + k * _CHUNK
            pltpu.sync_copy(accv.at[pl.ds(off, _CHUNK)], vals_0)

            @pl.when(cid == 0)
            def _():
                pltpu.sync_copy(vals_0, outv0_hbm.at[pl.ds(off, _CHUNK)])

            @pl.when(cid == 1)
            def _():
                pltpu.sync_copy(vals_0, outv1_hbm.at[pl.ds(off, _CHUNK)])


def _sc_scatter(vals, src, z128):
    fn = pl.kernel(
        _sc_scatter_body,
        out_type=(jax.ShapeDtypeStruct((_NPAD, HID), jnp.float32),
                  jax.ShapeDtypeStruct((_NPAD, HID), jnp.float32)),
        mesh=_sc_mesh(),
        scratch_types=[
            pltpu.VMEM((_CHUNK,), jnp.int32),
            pltpu.VMEM((_CHUNK,), jnp.int32),
            pltpu.VMEM((_CHUNK, HID), jnp.float32),
            pltpu.VMEM((_CHUNK, HID), jnp.float32),
            pltpu.SemaphoreType.DMA,
            pltpu.SemaphoreType.DMA,
            pltpu.VMEM_SHARED((_NPAD, HID), jnp.float32),
        ],
    )
    outv0, outv1 = fn(vals, src, z128)
    return jnp.stack([outv0[:N], outv1[:N]])


def _gelu(x):
    return x * 0.5 * (1.0 + lax.erf(x * (1.0 / math.sqrt(2.0))))


def _head_expand():
    col = lax.broadcasted_iota(jnp.int32, (HEADS, HID), 1) // DH
    row = lax.broadcasted_iota(jnp.int32, (HEADS, HID), 0)
    return (col == row).astype(jnp.float32)


def _edge_attn_kernel(he_ref, sc_ref, sh_ref, hs_ref, hd_ref,
                      wb1, bb1, wb2, bb2, wb3, bb3,
                      wv1, bv1, wv2, bv2, wv3, bv3,
                      evv_ref, ew_ref):
    he = he_ref[...] * sc_ref[...] + sh_ref[...]
    hs = hs_ref[...]
    hd = hd_ref[...]
    x384 = jnp.concatenate([hs, he, hd], axis=1)
    a = jnp.maximum(jnp.dot(x384, wb1[...], preferred_element_type=jnp.float32) + bb1[...], 0.0)
    a = jnp.maximum(jnp.dot(a, wb2[...], preferred_element_type=jnp.float32) + bb2[...], 0.0)
    w = jnp.dot(a, wb3[...], preferred_element_type=jnp.float32) + bb3[...]
    ew = jnp.exp(w * (1.0 / math.sqrt(DH)))
    x256 = jnp.concatenate([he, hd], axis=1)
    v = _gelu(jnp.dot(x256, wv1[...], preferred_element_type=jnp.float32) + bv1[...])
    v = _gelu(jnp.dot(v, wv2[...], preferred_element_type=jnp.float32) + bv2[...])
    v = jnp.dot(v, wv3[...], preferred_element_type=jnp.float32) + bv3[...]
    evv_ref[...] = v * jnp.dot(ew, _head_expand(), preferred_element_type=jnp.float32)
    ew_ref[...] = jnp.concatenate(
        [ew, jnp.zeros((ew.shape[0], HID - HEADS), jnp.float32)], axis=1)


def _edge_attn(he_raw, scale, shift, hs, hd, p):
    eb = lambda i: (i, 0)
    z2 = lambda i: (0, 0)
    in_specs = [
        pl.BlockSpec((_EB, HID), eb),
        pl.BlockSpec((1, HID), z2),
        pl.BlockSpec((1, HID), z2),
        pl.BlockSpec((_EB, HID), eb),
        pl.BlockSpec((_EB, HID), eb),
    ]
    weights = [p["B1"]["W"], p["B1"]["b"], p["B2"]["W"], p["B2"]["b"],
               p["B3"]["W"], p["B3"]["b"],
               p["WV1"]["W"], p["WV1"]["b"], p["WV2"]["W"], p["WV2"]["b"],
               p["WV3"]["W"], p["WV3"]["b"]]
    weights = [w if w.ndim == 2 else w.reshape(1, -1) for w in weights]
    in_specs += [pl.BlockSpec(w.shape, z2) for w in weights]
    return pl.pallas_call(
        _edge_attn_kernel,
        grid=(_EGRID,),
        in_specs=in_specs,
        out_specs=(pl.BlockSpec((_EB, HID), eb), pl.BlockSpec((_EB, HID), eb)),
        out_shape=(jax.ShapeDtypeStruct((E, HID), jnp.float32),
                   jax.ShapeDtypeStruct((E, HID), jnp.float32)),
    )(he_raw, scale, shift, hs, hd, *weights)


def _edge_mlp_kernel(he_ref, sc_ref, sh_ref, hs_ref, hd_ref,
                     we1, be1, we2, be2, we3, be3, g_ref, b_ref,
                     y_ref, sum_ref, sq_ref, osc_ref, osh_ref):
    i = pl.program_id(0)
    he = he_ref[...] * sc_ref[...] + sh_ref[...]
    x384 = jnp.concatenate([hs_ref[...], he, hd_ref[...]], axis=1)
    m = _gelu(jnp.dot(x384, we1[...], preferred_element_type=jnp.float32) + be1[...])
    m = _gelu(jnp.dot(m, we2[...], preferred_element_type=jnp.float32) + be2[...])
    m = jnp.dot(m, we3[...], preferred_element_type=jnp.float32) + be3[...]
    y = he + m
    y_ref[...] = y
    ps = jnp.sum(y, axis=0, keepdims=True)
    pq = jnp.sum(y * y, axis=0, keepdims=True)

    @pl.when(i == 0)
    def _():
        sum_ref[...] = ps
        sq_ref[...] = pq

    @pl.when(i > 0)
    def _():
        sum_ref[...] += ps
        sq_ref[...] += pq

    @pl.when(i == _EGRID - 1)
    def _():
        mean = sum_ref[...] * (1.0 / E)
        var = sq_ref[...] * (1.0 / E) - mean * mean
        sc = g_ref[...] * lax.rsqrt(var + 1e-5)
        osc_ref[...] = sc
        osh_ref[...] = b_ref[...] - mean * sc


def _edge_mlp(he_raw, scale, shift, hs, hd, p):
    eb = lambda i: (i, 0)
    z2 = lambda i: (0, 0)
    in_specs = [
        pl.BlockSpec((_EB, HID), eb),
        pl.BlockSpec((1, HID), z2),
        pl.BlockSpec((1, HID), z2),
        pl.BlockSpec((_EB, HID), eb),
        pl.BlockSpec((_EB, HID), eb),
    ]
    weights = [p["E1"]["W"], p["E1"]["b"], p["E2"]["W"], p["E2"]["b"],
               p["E3"]["W"], p["E3"]["b"], p["bne"]["g"], p["bne"]["b"]]
    weights = [w if w.ndim == 2 else w.reshape(1, -1) for w in weights]
    in_specs += [pl.BlockSpec(w.shape, z2) for w in weights]
    return pl.pallas_call(
        _edge_mlp_kernel,
        grid=(_EGRID,),
        in_specs=in_specs,
        out_specs=(pl.BlockSpec((_EB, HID), eb),
                   pl.BlockSpec((1, HID), z2), pl.BlockSpec((1, HID), z2),
                   pl.BlockSpec((1, HID), z2), pl.BlockSpec((1, HID), z2)),
        out_shape=(jax.ShapeDtypeStruct((E, HID), jnp.float32),
                   jax.ShapeDtypeStruct((1, HID), jnp.float32),
                   jax.ShapeDtypeStruct((1, HID), jnp.float32),
                   jax.ShapeDtypeStruct((1, HID), jnp.float32),
                   jax.ShapeDtypeStruct((1, HID), jnp.float32)),
    )(he_raw, scale, shift, hs, hd, *weights)


def _node_kernel(hvp_ref, denp_ref, hv_ref, bidr_ref, bidc_ref,
                 wo, g0, b0, wd1, bd1, wd2, bd2, g1, b1,
                 wg1, bg1, wg2, bg2, wg3, bg3, pre_ref, out_ref):
    hv = hvp_ref[0] + hvp_ref[1]
    den = denp_ref[0] + denp_ref[1]
    den4 = den[:, :HEADS]
    denx = jnp.dot(den4, _head_expand(), preferred_element_type=jnp.float32)
    hvn = hv / (denx + 1e-12)
    dh = jnp.dot(hvn, wo[...], preferred_element_type=jnp.float32)
    x = hv_ref[...] + dh
    m = jnp.mean(x, axis=0, keepdims=True)
    v = jnp.mean((x - m) * (x - m), axis=0, keepdims=True)
    x = (x - m) * lax.rsqrt(v + 1e-5) * g0[...] + b0[...]
    h = bd2[...] + jnp.zeros((N, HID), jnp.float32)
    for k in range(4):
        hk = jnp.maximum(
            jnp.dot(x, wd1[:, k * HID:(k + 1) * HID],
                    preferred_element_type=jnp.float32)
            + bd1[:, k * HID:(k + 1) * HID], 0.0)
        h = h + jnp.dot(hk, wd2[k * HID:(k + 1) * HID, :],
                        preferred_element_type=jnp.float32)
    x2 = x + h
    m2 = jnp.mean(x2, axis=0, keepdims=True)
    v2 = jnp.mean((x2 - m2) * (x2 - m2), axis=0, keepdims=True)
    x2 = (x2 - m2) * lax.rsqrt(v2 + 1e-5) * g1[...] + b1[...]
    pre_ref[...] = x2
    rows = lax.broadcasted_iota(jnp.int32, (NG, N), 0)
    oh = (rows == bidr_ref[...]).astype(jnp.float32)
    cnt = jnp.sum(oh, axis=1, keepdims=True)
    cv = jnp.dot(oh, x2, preferred_element_type=jnp.float32) / jnp.maximum(cnt, 1.0)
    gg = jnp.maximum(jnp.dot(cv, wg1[...], preferred_element_type=jnp.float32) + bg1[...], 0.0)
    gg = jnp.maximum(jnp.dot(gg, wg2[...], preferred_element_type=jnp.float32) + bg2[...], 0.0)
    gg = jax.nn.sigmoid(jnp.dot(gg, wg3[...], preferred_element_type=jnp.float32) + bg3[...])
    cols = lax.broadcasted_iota(jnp.int32, (N, NG), 1)
    oht = (cols == bidc_ref[...]).astype(jnp.float32)
    out_ref[...] = x2 * jnp.dot(oht, gg, preferred_element_type=jnp.float32)


def _node_update(hvp, denp, hv, bidr, bidc, p):
    weights = [p["WO"], p["bn0"]["g"], p["bn0"]["b"],
               p["D1"]["W"], p["D1"]["b"], p["D2"]["W"], p["D2"]["b"],
               p["bn1"]["g"], p["bn1"]["b"],
               p["G1"]["W"], p["G1"]["b"], p["G2"]["W"], p["G2"]["b"],
               p["G3"]["W"], p["G3"]["b"]]
    weights = [w if w.ndim == 2 else w.reshape(1, -1) for w in weights]
    return pl.pallas_call(
        _node_kernel,
        out_shape=(jax.ShapeDtypeStruct((N, HID), jnp.float32),
                   jax.ShapeDtypeStruct((N, HID), jnp.float32)),
    )(hvp, denp, hv, bidr, bidc, *weights)


def _finalize_kernel(y_ref, sc_ref, sh_ref, out_ref):
    out_ref[...] = y_ref[...] * sc_ref[...] + sh_ref[...]


def _finalize_edges(y, scale, shift):
    eb = lambda i: (i, 0)
    z2 = lambda i: (0, 0)
    return pl.pallas_call(
        _finalize_kernel,
        grid=(_EGRID,),
        in_specs=[pl.BlockSpec((_EB, HID), eb),
                  pl.BlockSpec((1, HID), z2), pl.BlockSpec((1, HID), z2)],
        out_specs=pl.BlockSpec((_EB, HID), eb),
        out_shape=jax.ShapeDtypeStruct((E, HID), jnp.float32),
    )(y, scale, shift)


def kernel(h_V, h_E, P_idx, batch_id, params):
    src = P_idx[0]
    dst = P_idx[1]
    bidr = batch_id.reshape(1, N)
    bidc = batch_id.reshape(N, 1)
    z128 = jnp.zeros((_NPAD, HID), jnp.float32)
    scale = jnp.ones((1, HID), jnp.float32)
    shift = jnp.zeros((1, HID), jnp.float32)
    he_raw = h_E
    hv = h_V
    for p in params:
        hs, hd = _sc_gather(hv, src, dst)
        evv, ewp = _edge_attn(he_raw, scale, shift, hs, hd, p)
        hvp = _sc_scatter(evv, src, z128)
        denp = _sc_scatter(ewp, src, z128)
        hv_pre, hv = _node_update(hvp, denp, hv, bidr, bidc, p)
        hs2, hd2 = _sc_gather(hv_pre, src, dst)
        he_raw, _s, _q, scale, shift = _edge_mlp(he_raw, scale, shift, hs2, hd2, p)
    he_out = _finalize_edges(he_raw, scale, shift)
    return (hv, he_out)


# speedup vs baseline: 25.2459x; 1.0002x over previous
import functools
import math

import jax
import jax.numpy as jnp
from jax import lax
from jax.experimental import pallas as pl
from jax.experimental.pallas import tpu as pltpu
from jax.experimental.pallas import tpu_sc as plsc

N = 10000
E = 320000
HID = 128
NIN = 256
HEADS = 4
DH = HID // HEADS
NG = 16

_CHUNK = 128
_SCH = 256
_NSUP = E // _SCH
_NW = 32
_STRIPS = (_NSUP + _NW - 1) // _NW
_NPAD = 10240
_RPS = _NPAD // 16
_EB = 1280
_EGRID = E // _EB

def _sc_mesh():
    return plsc.VectorSubcoreMesh(core_axis_name="c", subcore_axis_name="s",
                                  num_cores=2, num_subcores=16)


def _sc_gather_body(hv_hbm, src_hbm, dst_hbm, outs_hbm, outd_hbm,
                    idx_s0, idx_s1, idx_d0, idx_d1,
                    rows_s0, rows_s1, rows_d0, rows_d1,
                    sem_i, sem_g, sem_w):
    wid = lax.axis_index("s") * 2 + lax.axis_index("c")
    rows = (rows_s0, rows_s1, rows_d0, rows_d1)

    def body(i, carry):
        c = wid + _NW * i

        @pl.when(c < _NSUP)
        def _():
            base = c * _SCH
            cis = [
                pltpu.async_copy(src_hbm.at[pl.ds(base, _CHUNK)], idx_s0, sem_i),
                pltpu.async_copy(src_hbm.at[pl.ds(base + _CHUNK, _CHUNK)], idx_s1, sem_i),
                pltpu.async_copy(dst_hbm.at[pl.ds(base, _CHUNK)], idx_d0, sem_i),
                pltpu.async_copy(dst_hbm.at[pl.ds(base + _CHUNK, _CHUNK)], idx_d1, sem_i),
            ]

            @pl.when(i > 0)
            def _():
                for r in rows:
                    pltpu.make_async_copy(hv_hbm.at[pl.ds(0, _CHUNK)], r, sem_w).wait()

            for cp in cis:
                cp.wait()
            cps = [
                pltpu.async_copy(hv_hbm.at[idx_s0], rows_s0, sem_g),
                pltpu.async_copy(hv_hbm.at[idx_s1], rows_s1, sem_g),
                pltpu.async_copy(hv_hbm.at[idx_d0], rows_d0, sem_g),
                pltpu.async_copy(hv_hbm.at[idx_d1], rows_d1, sem_g),
            ]
            for cp in cps:
                cp.wait()
            pltpu.async_copy(rows_s0, outs_hbm.at[pl.ds(base, _CHUNK)], sem_w)
            pltpu.async_copy(rows_s1, outs_hbm.at[pl.ds(base + _CHUNK, _CHUNK)], sem_w)
            pltpu.async_copy(rows_d0, outd_hbm.at[pl.ds(base, _CHUNK)], sem_w)
            pltpu.async_copy(rows_d1, outd_hbm.at[pl.ds(base + _CHUNK, _CHUNK)], sem_w)
        return carry

    lax.fori_loop(0, _STRIPS, body, 0)
    for r in rows:
        pltpu.make_async_copy(hv_hbm.at[pl.ds(0, _CHUNK)], r, sem_w).wait()


def _sc_gather(h_v, src, dst):
    fn = pl.kernel(
        _sc_gather_body,
        out_type=(jax.ShapeDtypeStruct((E, HID), jnp.float32),
                  jax.ShapeDtypeStruct((E, HID), jnp.float32)),
        mesh=_sc_mesh(),
        scratch_types=[
            pltpu.VMEM((_CHUNK,), jnp.int32),
            pltpu.VMEM((_CHUNK,), jnp.int32),
            pltpu.VMEM((_CHUNK,), jnp.int32),
            pltpu.VMEM((_CHUNK,), jnp.int32),
            pltpu.VMEM((_CHUNK, HID), jnp.float32),
            pltpu.VMEM((_CHUNK, HID), jnp.float32),
            pltpu.VMEM((_CHUNK, HID), jnp.float32),
            pltpu.VMEM((_CHUNK, HID), jnp.float32),
            pltpu.SemaphoreType.DMA,
            pltpu.SemaphoreType.DMA,
            pltpu.SemaphoreType.DMA,
        ],
    )
    return fn(h_v, src, dst)


def _sc_scatter_body(vals_hbm, src_hbm, z128_hbm,
                     outv0_hbm, outv1_hbm, idx_0, idx_1, vals_0, vals_1,
                     sem_l, sem, accv):
    cid = lax.axis_index("c")
    sid = lax.axis_index("s")
    wid = sid * 2 + cid
    rbase = sid * _RPS

    pltpu.sync_copy(z128_hbm.at[pl.ds(0, _CHUNK)], vals_0)
    for k in range(_RPS // _CHUNK):
        pltpu.sync_copy(vals_0, accv.at[pl.ds(rbase + k * _CHUNK, _CHUNK)])
    plsc.subcore_barrier()

    def body(i, carry):
        c = wid + _NW * i

        @pl.when(c < _NSUP)
        def _():
            base = c * _SCH
            cls = [
                pltpu.async_copy(src_hbm.at[pl.ds(base, _CHUNK)], idx_0, sem_l),
                pltpu.async_copy(src_hbm.at[pl.ds(base + _CHUNK, _CHUNK)], idx_1, sem_l),
                pltpu.async_copy(vals_hbm.at[pl.ds(base, _CHUNK)], vals_0, sem_l),
                pltpu.async_copy(vals_hbm.at[pl.ds(base + _CHUNK, _CHUNK)], vals_1, sem_l),
            ]
            for cp in cls:
                cp.wait()
            cps = [
                pltpu.async_copy(vals_0, accv.at[idx_0], sem, add=True),
                pltpu.async_copy(vals_1, accv.at[idx_1], sem, add=True),
            ]
            for cp in cps:
                cp.wait()
        return carry

    lax.fori_loop(0, _STRIPS, body, 0)
    plsc.subcore_barrier()

    for k in range(_RPS // _CHUNK):
        off = rbase + k * _CHUNK
        pltpu.sync_copy(accv.at[pl.ds(off, _CHUNK)], vals_0)

        @pl.when(cid == 0)
        def _():
            pltpu.sync_copy(vals_0, outv0_hbm.at[pl.ds(off, _CHUNK)])

        @pl.when(cid == 1)
        def _():
            pltpu.sync_copy(vals_0, outv1_hbm.at[pl.ds(off, _CHUNK)])


def _sc_scatter(vals, src, z128):
    fn = pl.kernel(
        _sc_scatter_body,
        out_type=(jax.ShapeDtypeStruct((_NPAD, HID), jnp.float32),
                  jax.ShapeDtypeStruct((_NPAD, HID), jnp.float32)),
        mesh=_sc_mesh(),
        scratch_types=[
            pltpu.VMEM((_CHUNK,), jnp.int32),
            pltpu.VMEM((_CHUNK,), jnp.int32),
            pltpu.VMEM((_CHUNK, HID), jnp.float32),
            pltpu.VMEM((_CHUNK, HID), jnp.float32),
            pltpu.SemaphoreType.DMA,
            pltpu.SemaphoreType.DMA,
            pltpu.VMEM_SHARED((_NPAD, HID), jnp.float32),
        ],
    )
    outv0, outv1 = fn(vals, src, z128)
    return jnp.stack([outv0[:N], outv1[:N]])


def _gelu(x):
    return x * 0.5 * (1.0 + lax.erf(x * (1.0 / math.sqrt(2.0))))


def _head_expand():
    col = lax.broadcasted_iota(jnp.int32, (HEADS, HID), 1) // DH
    row = lax.broadcasted_iota(jnp.int32, (HEADS, HID), 0)
    return (col == row).astype(jnp.float32)


def _edge_attn_kernel(he_ref, sc_ref, sh_ref, hs_ref, hd_ref,
                      wb1, bb1, wb2, bb2, wb3, bb3,
                      wv1, bv1, wv2, bv2, wv3, bv3,
                      evv_ref, ew_ref):
    he = he_ref[...] * sc_ref[...] + sh_ref[...]
    hs = hs_ref[...]
    hd = hd_ref[...]
    x384 = jnp.concatenate([hs, he, hd], axis=1)
    a = jnp.maximum(jnp.dot(x384, wb1[...], preferred_element_type=jnp.float32) + bb1[...], 0.0)
    a = jnp.maximum(jnp.dot(a, wb2[...], preferred_element_type=jnp.float32) + bb2[...], 0.0)
    w = jnp.dot(a, wb3[...], preferred_element_type=jnp.float32) + bb3[...]
    ew = jnp.exp(w * (1.0 / math.sqrt(DH)))
    x256 = jnp.concatenate([he, hd], axis=1)
    v = _gelu(jnp.dot(x256, wv1[...], preferred_element_type=jnp.float32) + bv1[...])
    v = _gelu(jnp.dot(v, wv2[...], preferred_element_type=jnp.float32) + bv2[...])
    v = jnp.dot(v, wv3[...], preferred_element_type=jnp.float32) + bv3[...]
    evv_ref[...] = v * jnp.dot(ew, _head_expand(), preferred_element_type=jnp.float32)
    ew_ref[...] = jnp.concatenate(
        [ew, jnp.zeros((ew.shape[0], HID - HEADS), jnp.float32)], axis=1)


def _edge_attn(he_raw, scale, shift, hs, hd, p):
    eb = lambda i: (i, 0)
    z2 = lambda i: (0, 0)
    in_specs = [
        pl.BlockSpec((_EB, HID), eb),
        pl.BlockSpec((1, HID), z2),
        pl.BlockSpec((1, HID), z2),
        pl.BlockSpec((_EB, HID), eb),
        pl.BlockSpec((_EB, HID), eb),
    ]
    weights = [p["B1"]["W"], p["B1"]["b"], p["B2"]["W"], p["B2"]["b"],
               p["B3"]["W"], p["B3"]["b"],
               p["WV1"]["W"], p["WV1"]["b"], p["WV2"]["W"], p["WV2"]["b"],
               p["WV3"]["W"], p["WV3"]["b"]]
    weights = [w if w.ndim == 2 else w.reshape(1, -1) for w in weights]
    in_specs += [pl.BlockSpec(w.shape, z2) for w in weights]
    return pl.pallas_call(
        _edge_attn_kernel,
        grid=(_EGRID,),
        in_specs=in_specs,
        out_specs=(pl.BlockSpec((_EB, HID), eb), pl.BlockSpec((_EB, HID), eb)),
        out_shape=(jax.ShapeDtypeStruct((E, HID), jnp.float32),
                   jax.ShapeDtypeStruct((E, HID), jnp.float32)),
    )(he_raw, scale, shift, hs, hd, *weights)


def _edge_mlp_kernel(he_ref, sc_ref, sh_ref, hs_ref, hd_ref,
                     we1, be1, we2, be2, we3, be3, g_ref, b_ref,
                     y_ref, sum_ref, sq_ref, osc_ref, osh_ref):
    i = pl.program_id(0)
    he = he_ref[...] * sc_ref[...] + sh_ref[...]
    x384 = jnp.concatenate([hs_ref[...], he, hd_ref[...]], axis=1)
    m = _gelu(jnp.dot(x384, we1[...], preferred_element_type=jnp.float32) + be1[...])
    m = _gelu(jnp.dot(m, we2[...], preferred_element_type=jnp.float32) + be2[...])
    m = jnp.dot(m, we3[...], preferred_element_type=jnp.float32) + be3[...]
    y = he + m
    y_ref[...] = y
    ps = jnp.sum(y, axis=0, keepdims=True)
    pq = jnp.sum(y * y, axis=0, keepdims=True)

    @pl.when(i == 0)
    def _():
        sum_ref[...] = ps
        sq_ref[...] = pq

    @pl.when(i > 0)
    def _():
        sum_ref[...] += ps
        sq_ref[...] += pq

    @pl.when(i == _EGRID - 1)
    def _():
        mean = sum_ref[...] * (1.0 / E)
        var = sq_ref[...] * (1.0 / E) - mean * mean
        sc = g_ref[...] * lax.rsqrt(var + 1e-5)
        osc_ref[...] = sc
        osh_ref[...] = b_ref[...] - mean * sc


def _edge_mlp(he_raw, scale, shift, hs, hd, p):
    eb = lambda i: (i, 0)
    z2 = lambda i: (0, 0)
    in_specs = [
        pl.BlockSpec((_EB, HID), eb),
        pl.BlockSpec((1, HID), z2),
        pl.BlockSpec((1, HID), z2),
        pl.BlockSpec((_EB, HID), eb),
        pl.BlockSpec((_EB, HID), eb),
    ]
    weights = [p["E1"]["W"], p["E1"]["b"], p["E2"]["W"], p["E2"]["b"],
               p["E3"]["W"], p["E3"]["b"], p["bne"]["g"], p["bne"]["b"]]
    weights = [w if w.ndim == 2 else w.reshape(1, -1) for w in weights]
    in_specs += [pl.BlockSpec(w.shape, z2) for w in weights]
    return pl.pallas_call(
        _edge_mlp_kernel,
        grid=(_EGRID,),
        in_specs=in_specs,
        out_specs=(pl.BlockSpec((_EB, HID), eb),
                   pl.BlockSpec((1, HID), z2), pl.BlockSpec((1, HID), z2),
                   pl.BlockSpec((1, HID), z2), pl.BlockSpec((1, HID), z2)),
        out_shape=(jax.ShapeDtypeStruct((E, HID), jnp.float32),
                   jax.ShapeDtypeStruct((1, HID), jnp.float32),
                   jax.ShapeDtypeStruct((1, HID), jnp.float32),
                   jax.ShapeDtypeStruct((1, HID), jnp.float32),
                   jax.ShapeDtypeStruct((1, HID), jnp.float32)),
    )(he_raw, scale, shift, hs, hd, *weights)


def _node_kernel(hvp_ref, denp_ref, hv_ref, bidr_ref, bidc_ref,
                 wo, g0, b0, wd1, bd1, wd2, bd2, g1, b1,
                 wg1, bg1, wg2, bg2, wg3, bg3, pre_ref, out_ref):
    hv = hvp_ref[0] + hvp_ref[1]
    den = denp_ref[0] + denp_ref[1]
    den4 = den[:, :HEADS]
    denx = jnp.dot(den4, _head_expand(), preferred_element_type=jnp.float32)
    hvn = hv / (denx + 1e-12)
    dh = jnp.dot(hvn, wo[...], preferred_element_type=jnp.float32)
    x = hv_ref[...] + dh
    m = jnp.mean(x, axis=0, keepdims=True)
    v = jnp.mean((x - m) * (x - m), axis=0, keepdims=True)
    x = (x - m) * lax.rsqrt(v + 1e-5) * g0[...] + b0[...]
    h = bd2[...] + jnp.zeros((N, HID), jnp.float32)
    for k in range(4):
        hk = jnp.maximum(
            jnp.dot(x, wd1[:, k * HID:(k + 1) * HID],
                    preferred_element_type=jnp.float32)
            + bd1[:, k * HID:(k + 1) * HID], 0.0)
        h = h + jnp.dot(hk, wd2[k * HID:(k + 1) * HID, :],
                        preferred_element_type=jnp.float32)
    x2 = x + h
    m2 = jnp.mean(x2, axis=0, keepdims=True)
    v2 = jnp.mean((x2 - m2) * (x2 - m2), axis=0, keepdims=True)
    x2 = (x2 - m2) * lax.rsqrt(v2 + 1e-5) * g1[...] + b1[...]
    pre_ref[...] = x2
    rows = lax.broadcasted_iota(jnp.int32, (NG, N), 0)
    oh = (rows == bidr_ref[...]).astype(jnp.float32)
    cnt = jnp.sum(oh, axis=1, keepdims=True)
    cv = jnp.dot(oh, x2, preferred_element_type=jnp.float32) / jnp.maximum(cnt, 1.0)
    gg = jnp.maximum(jnp.dot(cv, wg1[...], preferred_element_type=jnp.float32) + bg1[...], 0.0)
    gg = jnp.maximum(jnp.dot(gg, wg2[...], preferred_element_type=jnp.float32) + bg2[...], 0.0)
    gg = jax.nn.sigmoid(jnp.dot(gg, wg3[...], preferred_element_type=jnp.float32) + bg3[...])
    cols = lax.broadcasted_iota(jnp.int32, (N, NG), 1)
    oht = (cols == bidc_ref[...]).astype(jnp.float32)
    out_ref[...] = x2 * jnp.dot(oht, gg, preferred_element_type=jnp.float32)


def _node_update(hvp, denp, hv, bidr, bidc, p):
    weights = [p["WO"], p["bn0"]["g"], p["bn0"]["b"],
               p["D1"]["W"], p["D1"]["b"], p["D2"]["W"], p["D2"]["b"],
               p["bn1"]["g"], p["bn1"]["b"],
               p["G1"]["W"], p["G1"]["b"], p["G2"]["W"], p["G2"]["b"],
               p["G3"]["W"], p["G3"]["b"]]
    weights = [w if w.ndim == 2 else w.reshape(1, -1) for w in weights]
    return pl.pallas_call(
        _node_kernel,
        out_shape=(jax.ShapeDtypeStruct((N, HID), jnp.float32),
                   jax.ShapeDtypeStruct((N, HID), jnp.float32)),
    )(hvp, denp, hv, bidr, bidc, *weights)


def _finalize_kernel(y_ref, sc_ref, sh_ref, out_ref):
    out_ref[...] = y_ref[...] * sc_ref[...] + sh_ref[...]


def _finalize_edges(y, scale, shift):
    eb = lambda i: (i, 0)
    z2 = lambda i: (0, 0)
    return pl.pallas_call(
        _finalize_kernel,
        grid=(_EGRID,),
        in_specs=[pl.BlockSpec((_EB, HID), eb),
                  pl.BlockSpec((1, HID), z2), pl.BlockSpec((1, HID), z2)],
        out_specs=pl.BlockSpec((_EB, HID), eb),
        out_shape=jax.ShapeDtypeStruct((E, HID), jnp.float32),
    )(y, scale, shift)


def kernel(h_V, h_E, P_idx, batch_id, params):
    src = P_idx[0]
    dst = P_idx[1]
    bidr = batch_id.reshape(1, N)
    bidc = batch_id.reshape(N, 1)
    z128 = jnp.zeros((_NPAD, HID), jnp.float32)
    scale = jnp.ones((1, HID), jnp.float32)
    shift = jnp.zeros((1, HID), jnp.float32)
    he_raw = h_E
    hv = h_V
    for p in params:
        hs, hd = _sc_gather(hv, src, dst)
        evv, ewp = _edge_attn(he_raw, scale, shift, hs, hd, p)
        hvp = _sc_scatter(evv, src, z128)
        denp = _sc_scatter(ewp, src, z128)
        hv_pre, hv = _node_update(hvp, denp, hv, bidr, bidc, p)
        hs2, hd2 = _sc_gather(hv_pre, src, dst)
        he_raw, _s, _q, scale, shift = _edge_mlp(he_raw, scale, shift, hs2, hd2, p)
    he_out = _finalize_edges(he_raw, scale, shift)
    return (hv, he_out)
